# baseline (speedup 1.0000x reference)
import math

import jax
import jax.numpy as jnp
from jax import lax
from jax.experimental import pallas as pl
from jax.experimental.pallas import tpu as pltpu
from jax.experimental.pallas import tpu_sc as plsc

VX, VY, VZ = 0.1, 0.1, 0.25
XMIN, XMAX = -10.0, 10.0
YMIN, YMAX = -6.0, 6.0
ZMIN, ZMAX = 0.3, 20.0
NX = int(math.floor((XMAX - XMIN) / VX)) + 1
NY = int(math.floor((YMAX - YMIN) / VY)) + 1
NZ = int(math.floor((ZMAX - ZMIN) / VZ)) + 1
NVOX = NX * NY * NZ

B = 4
P = 512 * 512
R2 = P // 128

BIGKEY = 1 << 29

NCHUNK = 2
C = 1044480
SPREAD = 2048
ZERO_BASE = C
DUMP_BASE = C + SPREAD
R_TOTAL = 1 << 20
STRIPE = R_TOTAL // 16

TPTS = P // 16
HALF = TPTS // 2
ZROWS = 8192


_RB = 128


def _prep_body(xyz_ref, rel_ref, geom_ref, key_ref, m_ref, mz_ref,
               mr_ref, mg_ref):
    x = xyz_ref[0, 0]
    y = xyz_ref[0, 1]
    z = xyz_ref[0, 2]
    r = rel_ref[0]
    g = geom_ref[0]
    keep = ((x >= XMIN) & (x <= XMAX)
            & (y >= YMIN) & (y <= YMAX)
            & (z >= ZMIN) & (z <= ZMAX)
            & (r > 0))
    ix = jnp.clip(jnp.floor((x - XMIN) / VX).astype(jnp.int32), 0, NX - 1)
    iy = jnp.clip(jnp.floor((y - YMIN) / VY).astype(jnp.int32), 0, NY - 1)
    iz = jnp.clip(jnp.floor((z - ZMIN) / VZ).astype(jnp.int32), 0, NZ - 1)
    key = (ix * NY + iy) * NZ + iz
    key_ref[0] = jnp.where(keep, key, BIGKEY)
    m = keep.astype(jnp.float32)
    m_ref[0] = m
    mz_ref[0] = z * m
    mr_ref[0] = r * m
    mg_ref[0] = g * m


def _prep(xyz_f, rel_f, geom_f):
    nb = R2 // _RB
    bs = pl.BlockSpec((1, _RB, 128), lambda b, n: (b, n, 0))
    return pl.pallas_call(
        _prep_body,
        grid=(B, nb),
        in_specs=[
            pl.BlockSpec((1, 3, _RB, 128), lambda b, n: (b, 0, n, 0)),
            bs, bs,
        ],
        out_specs=[bs, bs, bs, bs, bs],
        out_shape=[
            jax.ShapeDtypeStruct((B, R2, 128), jnp.int32),
            jax.ShapeDtypeStruct((B, R2, 128), jnp.float32),
            jax.ShapeDtypeStruct((B, R2, 128), jnp.float32),
            jax.ShapeDtypeStruct((B, R2, 128), jnp.float32),
            jax.ShapeDtypeStruct((B, R2, 128), jnp.float32),
        ],
    )(xyz_f, rel_f, geom_f)


def _sc_body(key_hbm, v0_hbm, v1_hbm, v2_hbm, v3_hbm, zer_hbm,
             *refs):
    outs_hbm = refs[:8]
    (lidxA, lidxB, gidxA, gidxB, wvA, wvB, zbuf_v, table,
     zsem, dsemA, dsemB) = refs[8:]
    vals_hbm = (v0_hbm, v1_hbm, v2_hbm, v3_hbm)
    core = lax.axis_index("c")
    sub = lax.axis_index("s")
    iota = lax.iota(jnp.int32, 16)

    pltpu.sync_copy(zer_hbm, zbuf_v)

    row0 = sub * STRIPE
    nz = STRIPE // ZROWS

    def fire_zeros():
        return [
            pltpu.async_copy(
                zbuf_v, table.at[pl.ds(row0 + zi * ZROWS, ZROWS)], zsem)
            for zi in range(nz)
        ]

    zcopies = fire_zeros()

    for bb in range(2):
        batch = 2 * core + bb
        pstart = batch * P + sub * TPTS

        for chunk in range(NCHUNK):
            base = chunk * C

            pltpu.sync_copy(key_hbm.at[pl.ds(pstart, HALF)], lidxA)
            pltpu.sync_copy(key_hbm.at[pl.ds(pstart + HALF, HALF)], lidxB)

            for ho, li, gi in ((0, lidxA, gidxA), (HALF, lidxB, gidxB)):
                @pl.loop(0, HALF // 16)
                def _idx(j, li=li, gi=gi, ho=ho):
                    off = j * 16
                    t = li[pl.ds(off, 16)] - base
                    inb = plsc.bitcast(t, jnp.uint32) < jnp.uint32(C)
                    sp = iota + ((off + ho) & (SPREAD - 1))
                    zvec = sp + ZERO_BASE
                    dvec = zvec + SPREAD
                    li[pl.ds(off, 16)] = jnp.where(inb, t, dvec)
                    gi[pl.ds(off, 16)] = jnp.where(inb, t, zvec)

            for stat in range(4):
                o = chunk * 4 + stat
                dA = pltpu.async_copy(
                    vals_hbm[stat].at[pl.ds(pstart, HALF)], wvA, dsemA)
                dB = pltpu.async_copy(
                    vals_hbm[stat].at[pl.ds(pstart + HALF, HALF)], wvB, dsemB)
                for zc in zcopies:
                    zc.wait()
                plsc.subcore_barrier()
                dA.wait()
                pltpu.sync_copy(wvA, table.at[lidxA], add=True)
                dB.wait()
                pltpu.sync_copy(wvB, table.at[lidxB], add=True)
                plsc.subcore_barrier()

                pltpu.sync_copy(table.at[gidxA], wvA)
                oA = pltpu.async_copy(
                    wvA, outs_hbm[o].at[pl.ds(pstart, HALF)], dsemA)
                pltpu.sync_copy(table.at[gidxB], wvB)
                plsc.subcore_barrier()
                zcopies = fire_zeros()
                oB = pltpu.async_copy(
                    wvB, outs_hbm[o].at[pl.ds(pstart + HALF, HALF)], dsemB)
                oA.wait()
                oB.wait()

    for zc in zcopies:
        zc.wait()


def _sc_aggregate(key_enc, m, mz, mr, mg, zer):
    mesh = plsc.VectorSubcoreMesh(core_axis_name="c", subcore_axis_name="s",
                                  num_cores=2, num_subcores=16)
    pt = jax.ShapeDtypeStruct((B * P,), jnp.float32)
    return pl.kernel(
        _sc_body,
        out_type=[pt] * 8,
        mesh=mesh,
        scratch_types=[
            pltpu.VMEM((HALF,), jnp.int32),
            pltpu.VMEM((HALF,), jnp.int32),
            pltpu.VMEM((HALF,), jnp.int32),
            pltpu.VMEM((HALF,), jnp.int32),
            pltpu.VMEM((HALF,), jnp.float32),
            pltpu.VMEM((HALF,), jnp.float32),
            pltpu.VMEM((ZROWS,), jnp.float32),
            pltpu.VMEM_SHARED((R_TOTAL,), jnp.float32),
            pltpu.SemaphoreType.DMA,
            pltpu.SemaphoreType.DMA,
            pltpu.SemaphoreType.DMA,
        ],
    )(key_enc, m, mz, mr, mg, zer)


_RF = 256
_NBF = R2 // _RF
_LOG32 = math.log(32.0)


def _score_body(c0_ref, z0_ref, r0_ref, g0_ref,
                c1_ref, z1_ref, r1_ref, g1_ref, m_ref,
                score_ref, pmin_ref, pmax_ref):
    cnt = c0_ref[0] + c1_ref[0]
    zs = z0_ref[0] + z1_ref[0]
    rs = r0_ref[0] + r1_ref[0]
    gs = g0_ref[0] + g1_ref[0]
    cc = jnp.maximum(cnt, 1.0)
    inv = 1.0 / cc
    z_mean = zs * inv
    rel_mean = rs * inv
    geom_mean = gs * inv
    density = jnp.log1p(cnt) / _LOG32
    range_score = jnp.exp(-0.08 * z_mean)
    vs = jnp.maximum(
        (0.55 * geom_mean + 0.25 * density + 0.2 * range_score) * rel_mean,
        0.0)
    score = vs * m_ref[0]
    score_ref[0] = score
    mn = jnp.min(score, axis=0)
    mx = jnp.max(score, axis=0)
    pmin_ref[0, 0] = jnp.broadcast_to(mn[None, :], (8, 128))
    pmax_ref[0, 0] = jnp.broadcast_to(mx[None, :], (8, 128))


def _score(halves, m):
    bs = pl.BlockSpec((1, _RF, 128), lambda b, n: (b, n, 0))
    return pl.pallas_call(
        _score_body,
        grid=(B, _NBF),
        in_specs=[bs] * 9,
        out_specs=[
            pl.BlockSpec((1, _RF, 128), lambda b, n: (b, n, 0)),
            pl.BlockSpec((1, 1, 8, 128), lambda b, n: (b, n, 0, 0)),
            pl.BlockSpec((1, 1, 8, 128), lambda b, n: (b, n, 0, 0)),
        ],
        out_shape=[
            jax.ShapeDtypeStruct((B, R2, 128), jnp.float32),
            jax.ShapeDtypeStruct((B, _NBF, 8, 128), jnp.float32),
            jax.ShapeDtypeStruct((B, _NBF, 8, 128), jnp.float32),
        ],
    )(*halves, m)


def _norm_body(score_ref, pmin_ref, pmax_ref, out_ref):
    mn = jnp.min(pmin_ref[0])
    mx = jnp.max(pmax_ref[0])
    out_ref[0] = (score_ref[0] - mn) / (mx - mn + 1e-6)


def _norm(score, pmin, pmax):
    return pl.pallas_call(
        _norm_body,
        grid=(B, _NBF),
        in_specs=[
            pl.BlockSpec((1, _RF, 128), lambda b, n: (b, n, 0)),
            pl.BlockSpec((1, _NBF, 8, 128), lambda b, n: (b, 0, 0, 0)),
            pl.BlockSpec((1, _NBF, 8, 128), lambda b, n: (b, 0, 0, 0)),
        ],
        out_specs=pl.BlockSpec((1, _RF, 128), lambda b, n: (b, n, 0)),
        out_shape=jax.ShapeDtypeStruct((B, R2, 128), jnp.float32),
    )(score, pmin, pmax)


def kernel(xyz, reliability, geom_break):
    xyz_f = xyz.reshape(B, 3, R2, 128)
    rel_f = reliability.reshape(B, R2, 128)
    geom_f = geom_break.reshape(B, R2, 128)

    key_enc, m, mz, mr, mg = _prep(xyz_f, rel_f, geom_f)
    zer = jnp.zeros((ZROWS,), jnp.float32)

    halves = _sc_aggregate(
        key_enc.reshape(B * P), m.reshape(B * P), mz.reshape(B * P),
        mr.reshape(B * P), mg.reshape(B * P), zer)

    score, pmin, pmax = _score([h.reshape(B, R2, 128) for h in halves], m)
    out = _norm(score, pmin, pmax)
    return out.reshape(B, 1, 512, 512)


# speedup vs baseline: 1.0922x; 1.0922x over previous
import math

import jax
import jax.numpy as jnp
from jax import lax
from jax.experimental import pallas as pl
from jax.experimental.pallas import tpu as pltpu
from jax.experimental.pallas import tpu_sc as plsc

VX, VY, VZ = 0.1, 0.1, 0.25
XMIN, XMAX = -10.0, 10.0
YMIN, YMAX = -6.0, 6.0
ZMIN, ZMAX = 0.3, 20.0
NX = int(math.floor((XMAX - XMIN) / VX)) + 1
NY = int(math.floor((YMAX - YMIN) / VY)) + 1
NZ = int(math.floor((ZMAX - ZMIN) / VZ)) + 1
NVOX = NX * NY * NZ

B = 4
P = 512 * 512
R2 = P // 128

BIGKEY = 1 << 29

NCHUNK = 2
C = 1044480
SPREAD = 2048
ZERO_BASE = C
DUMP_BASE = C + SPREAD
R_TOTAL = 1 << 20
STRIPE = R_TOTAL // 16

TPTS = P // 16
HALF = TPTS // 2
ZROWS = 8192


_RB = 128


def _prep_body(xyz_ref, rel_ref, geom_ref, key_ref, m_ref, mz_ref,
               mr_ref, mg_ref):
    x = xyz_ref[0, 0]
    y = xyz_ref[0, 1]
    z = xyz_ref[0, 2]
    r = rel_ref[0]
    g = geom_ref[0]
    keep = ((x >= XMIN) & (x <= XMAX)
            & (y >= YMIN) & (y <= YMAX)
            & (z >= ZMIN) & (z <= ZMAX)
            & (r > 0))
    ix = jnp.clip(jnp.floor((x - XMIN) / VX).astype(jnp.int32), 0, NX - 1)
    iy = jnp.clip(jnp.floor((y - YMIN) / VY).astype(jnp.int32), 0, NY - 1)
    iz = jnp.clip(jnp.floor((z - ZMIN) / VZ).astype(jnp.int32), 0, NZ - 1)
    key = (ix * NY + iy) * NZ + iz
    key_ref[0] = jnp.where(keep, key, BIGKEY)
    m = keep.astype(jnp.float32)
    m_ref[0] = m
    mz_ref[0] = z * m
    mr_ref[0] = r * m
    mg_ref[0] = g * m


def _prep(xyz_f, rel_f, geom_f, off):
    nb = R2 // _RB
    ibs = pl.BlockSpec((1, _RB, 128), lambda b, n: (b + off, n, 0))
    obs = pl.BlockSpec((1, _RB, 128), lambda b, n: (b, n, 0))
    return pl.pallas_call(
        _prep_body,
        grid=(2, nb),
        in_specs=[
            pl.BlockSpec((1, 3, _RB, 128), lambda b, n: (b + off, 0, n, 0)),
            ibs, ibs,
        ],
        out_specs=[obs] * 5,
        out_shape=[
            jax.ShapeDtypeStruct((2, R2, 128), jnp.int32),
            jax.ShapeDtypeStruct((2, R2, 128), jnp.float32),
            jax.ShapeDtypeStruct((2, R2, 128), jnp.float32),
            jax.ShapeDtypeStruct((2, R2, 128), jnp.float32),
            jax.ShapeDtypeStruct((2, R2, 128), jnp.float32),
        ],
    )(xyz_f, rel_f, geom_f)


def _sc_body(key_hbm, v0_hbm, v1_hbm, v2_hbm, v3_hbm, zer_hbm,
             *refs):
    outs_hbm = refs[:8]
    (lidxA, lidxB, gidxA, gidxB, wvA, wvB, zbuf_v, table,
     zsem, dsemA, dsemB) = refs[8:]
    vals_hbm = (v0_hbm, v1_hbm, v2_hbm, v3_hbm)
    core = lax.axis_index("c")
    sub = lax.axis_index("s")
    iota = lax.iota(jnp.int32, 16)

    pltpu.sync_copy(zer_hbm, zbuf_v)

    row0 = sub * STRIPE
    nz = STRIPE // ZROWS

    def fire_zeros():
        return [
            pltpu.async_copy(
                zbuf_v, table.at[pl.ds(row0 + zi * ZROWS, ZROWS)], zsem)
            for zi in range(nz)
        ]

    zcopies = fire_zeros()

    if True:
        pstart = core * P + sub * TPTS
        ostart = pstart

        for chunk in range(NCHUNK):
            base = chunk * C

            pltpu.sync_copy(key_hbm.at[pl.ds(pstart, HALF)], lidxA)
            pltpu.sync_copy(key_hbm.at[pl.ds(pstart + HALF, HALF)], lidxB)

            for ho, li, gi in ((0, lidxA, gidxA), (HALF, lidxB, gidxB)):
                @pl.loop(0, HALF // 16)
                def _idx(j, li=li, gi=gi, ho=ho):
                    off = j * 16
                    t = li[pl.ds(off, 16)] - base
                    inb = plsc.bitcast(t, jnp.uint32) < jnp.uint32(C)
                    sp = iota + ((off + ho) & (SPREAD - 1))
                    zvec = sp + ZERO_BASE
                    dvec = zvec + SPREAD
                    li[pl.ds(off, 16)] = jnp.where(inb, t, dvec)
                    gi[pl.ds(off, 16)] = jnp.where(inb, t, zvec)

            for stat in range(4):
                o = chunk * 4 + stat
                dA = pltpu.async_copy(
                    vals_hbm[stat].at[pl.ds(pstart, HALF)], wvA, dsemA)
                dB = pltpu.async_copy(
                    vals_hbm[stat].at[pl.ds(pstart + HALF, HALF)], wvB, dsemB)
                for zc in zcopies:
                    zc.wait()
                plsc.subcore_barrier()
                dA.wait()
                pltpu.sync_copy(wvA, table.at[lidxA], add=True)
                dB.wait()
                pltpu.sync_copy(wvB, table.at[lidxB], add=True)
                plsc.subcore_barrier()

                pltpu.sync_copy(table.at[gidxA], wvA)
                oA = pltpu.async_copy(
                    wvA, outs_hbm[o].at[pl.ds(ostart, HALF)], dsemA)
                pltpu.sync_copy(table.at[gidxB], wvB)
                plsc.subcore_barrier()
                zcopies = fire_zeros()
                oB = pltpu.async_copy(
                    wvB, outs_hbm[o].at[pl.ds(ostart + HALF, HALF)], dsemB)
                oA.wait()
                oB.wait()

    for zc in zcopies:
        zc.wait()


def _sc_aggregate(key_enc, m, mz, mr, mg, zer):
    mesh = plsc.VectorSubcoreMesh(core_axis_name="c", subcore_axis_name="s",
                                  num_cores=2, num_subcores=16)
    pt = jax.ShapeDtypeStruct((2 * P,), jnp.float32)
    return pl.kernel(
        _sc_body,
        out_type=[pt] * 8,
        mesh=mesh,
        scratch_types=[
            pltpu.VMEM((HALF,), jnp.int32),
            pltpu.VMEM((HALF,), jnp.int32),
            pltpu.VMEM((HALF,), jnp.int32),
            pltpu.VMEM((HALF,), jnp.int32),
            pltpu.VMEM((HALF,), jnp.float32),
            pltpu.VMEM((HALF,), jnp.float32),
            pltpu.VMEM((ZROWS,), jnp.float32),
            pltpu.VMEM_SHARED((R_TOTAL,), jnp.float32),
            pltpu.SemaphoreType.DMA,
            pltpu.SemaphoreType.DMA,
            pltpu.SemaphoreType.DMA,
        ],
    )(key_enc, m, mz, mr, mg, zer)


_RF = 256
_NBF = R2 // _RF
_LOG32 = math.log(32.0)


def _score_body(c0_ref, z0_ref, r0_ref, g0_ref,
                c1_ref, z1_ref, r1_ref, g1_ref, m_ref,
                score_ref, pmin_ref, pmax_ref):
    cnt = c0_ref[0] + c1_ref[0]
    zs = z0_ref[0] + z1_ref[0]
    rs = r0_ref[0] + r1_ref[0]
    gs = g0_ref[0] + g1_ref[0]
    cc = jnp.maximum(cnt, 1.0)
    inv = 1.0 / cc
    z_mean = zs * inv
    rel_mean = rs * inv
    geom_mean = gs * inv
    density = jnp.log1p(cnt) / _LOG32
    range_score = jnp.exp(-0.08 * z_mean)
    vs = jnp.maximum(
        (0.55 * geom_mean + 0.25 * density + 0.2 * range_score) * rel_mean,
        0.0)
    score = vs * m_ref[0]
    score_ref[0] = score
    mn = jnp.min(score, axis=0)
    mx = jnp.max(score, axis=0)
    pmin_ref[0, 0] = jnp.broadcast_to(mn[None, :], (8, 128))
    pmax_ref[0, 0] = jnp.broadcast_to(mx[None, :], (8, 128))


def _score(halves, m):
    bs = pl.BlockSpec((1, _RF, 128), lambda b, n: (b, n, 0))
    return pl.pallas_call(
        _score_body,
        grid=(2, _NBF),
        in_specs=[bs] * 9,
        out_specs=[
            pl.BlockSpec((1, _RF, 128), lambda b, n: (b, n, 0)),
            pl.BlockSpec((1, 1, 8, 128), lambda b, n: (b, n, 0, 0)),
            pl.BlockSpec((1, 1, 8, 128), lambda b, n: (b, n, 0, 0)),
        ],
        out_shape=[
            jax.ShapeDtypeStruct((2, R2, 128), jnp.float32),
            jax.ShapeDtypeStruct((2, _NBF, 8, 128), jnp.float32),
            jax.ShapeDtypeStruct((2, _NBF, 8, 128), jnp.float32),
        ],
    )(*halves, m)


def _norm_body(score_ref, pmin_ref, pmax_ref, out_ref):
    mn = jnp.min(pmin_ref[0])
    mx = jnp.max(pmax_ref[0])
    out_ref[0] = (score_ref[0] - mn) / (mx - mn + 1e-6)


def _norm(score, pmin, pmax):
    return pl.pallas_call(
        _norm_body,
        grid=(2, _NBF),
        in_specs=[
            pl.BlockSpec((1, _RF, 128), lambda b, n: (b, n, 0)),
            pl.BlockSpec((1, _NBF, 8, 128), lambda b, n: (b, 0, 0, 0)),
            pl.BlockSpec((1, _NBF, 8, 128), lambda b, n: (b, 0, 0, 0)),
        ],
        out_specs=pl.BlockSpec((1, _RF, 128), lambda b, n: (b, n, 0)),
        out_shape=jax.ShapeDtypeStruct((2, R2, 128), jnp.float32),
    )(score, pmin, pmax)


def kernel(xyz, reliability, geom_break):
    xyz_f = xyz.reshape(B, 3, R2, 128)
    rel_f = reliability.reshape(B, R2, 128)
    geom_f = geom_break.reshape(B, R2, 128)

    zer = jnp.zeros((ZROWS,), jnp.float32)

    outs = []
    for off in (0, 2):
        key_enc, m, mz, mr, mg = _prep(xyz_f, rel_f, geom_f, off)
        halves = _sc_aggregate(
            key_enc.reshape(2 * P), m.reshape(2 * P), mz.reshape(2 * P),
            mr.reshape(2 * P), mg.reshape(2 * P), zer)
        score, pmin, pmax = _score(
            [h.reshape(2, R2, 128) for h in halves], m)
        outs.append(_norm(score, pmin, pmax))

    return jnp.concatenate(outs, axis=0).reshape(B, 1, 512, 512)


# speedup vs baseline: 1.1106x; 1.0169x over previous
import math

import jax
import jax.numpy as jnp
from jax import lax
from jax.experimental import pallas as pl
from jax.experimental.pallas import tpu as pltpu
from jax.experimental.pallas import tpu_sc as plsc

VX, VY, VZ = 0.1, 0.1, 0.25
XMIN, XMAX = -10.0, 10.0
YMIN, YMAX = -6.0, 6.0
ZMIN, ZMAX = 0.3, 20.0
NX = int(math.floor((XMAX - XMIN) / VX)) + 1
NY = int(math.floor((YMAX - YMIN) / VY)) + 1
NZ = int(math.floor((ZMAX - ZMIN) / VZ)) + 1
NVOX = NX * NY * NZ

B = 4
P = 512 * 512
R2 = P // 128

BIGKEY = 1 << 29

NCHUNK = 2
C = 1044480
SPREAD = 2048
ZERO_BASE = C
DUMP_BASE = C + SPREAD
R_TOTAL = 1 << 20
STRIPE = R_TOTAL // 16

TPTS = P // 16
HALF = TPTS // 2
ZROWS = 8192


_RB = 128


def _prep_body(xyz_ref, rel_ref, geom_ref, key_ref, m_ref, mz_ref,
               mr_ref, mg_ref):
    x = xyz_ref[0, 0]
    y = xyz_ref[0, 1]
    z = xyz_ref[0, 2]
    r = rel_ref[0]
    g = geom_ref[0]
    keep = ((x >= XMIN) & (x <= XMAX)
            & (y >= YMIN) & (y <= YMAX)
            & (z >= ZMIN) & (z <= ZMAX)
            & (r > 0))
    ix = jnp.clip(jnp.floor((x - XMIN) / VX).astype(jnp.int32), 0, NX - 1)
    iy = jnp.clip(jnp.floor((y - YMIN) / VY).astype(jnp.int32), 0, NY - 1)
    iz = jnp.clip(jnp.floor((z - ZMIN) / VZ).astype(jnp.int32), 0, NZ - 1)
    key = (ix * NY + iy) * NZ + iz
    key_ref[0] = jnp.where(keep, key, BIGKEY)
    m = keep.astype(jnp.float32)
    m_ref[0] = m
    mz_ref[0] = z * m
    mr_ref[0] = r * m
    mg_ref[0] = g * m


def _prep(xyz_f, rel_f, geom_f, off):
    nb = R2 // _RB
    ibs = pl.BlockSpec((1, _RB, 128), lambda n: (off, n, 0))
    obs = pl.BlockSpec((1, _RB, 128), lambda n: (0, n, 0))
    return pl.pallas_call(
        _prep_body,
        grid=(nb,),
        in_specs=[
            pl.BlockSpec((1, 3, _RB, 128), lambda n: (off, 0, n, 0)),
            ibs, ibs,
        ],
        out_specs=[obs] * 5,
        out_shape=[
            jax.ShapeDtypeStruct((1, R2, 128), jnp.int32),
            jax.ShapeDtypeStruct((1, R2, 128), jnp.float32),
            jax.ShapeDtypeStruct((1, R2, 128), jnp.float32),
            jax.ShapeDtypeStruct((1, R2, 128), jnp.float32),
            jax.ShapeDtypeStruct((1, R2, 128), jnp.float32),
        ],
    )(xyz_f, rel_f, geom_f)


def _sc_body(key_hbm, v0_hbm, v1_hbm, v2_hbm, v3_hbm, zer_hbm,
             *refs):
    outs_hbm = refs[:4]
    (lidxA, lidxB, gidxA, gidxB, wvA, wvB, zbuf_v, table,
     zsem, dsemA, dsemB) = refs[4:]
    vals_hbm = (v0_hbm, v1_hbm, v2_hbm, v3_hbm)
    core = lax.axis_index("c")
    sub = lax.axis_index("s")
    iota = lax.iota(jnp.int32, 16)

    pltpu.sync_copy(zer_hbm, zbuf_v)

    row0 = sub * STRIPE
    nz = STRIPE // ZROWS

    def fire_zeros():
        return [
            pltpu.async_copy(
                zbuf_v, table.at[pl.ds(row0 + zi * ZROWS, ZROWS)], zsem)
            for zi in range(nz)
        ]

    zcopies = fire_zeros()

    if True:
        pstart = sub * TPTS
        ostart = core * P + sub * TPTS

        for chunk_ in range(1):
            base = core * C

            pltpu.sync_copy(key_hbm.at[pl.ds(pstart, HALF)], lidxA)
            pltpu.sync_copy(key_hbm.at[pl.ds(pstart + HALF, HALF)], lidxB)

            for ho, li, gi in ((0, lidxA, gidxA), (HALF, lidxB, gidxB)):
                @pl.loop(0, HALF // 16)
                def _idx(j, li=li, gi=gi, ho=ho):
                    off = j * 16
                    t = li[pl.ds(off, 16)] - base
                    inb = plsc.bitcast(t, jnp.uint32) < jnp.uint32(C)
                    sp = iota + ((off + ho) & (SPREAD - 1))
                    zvec = sp + ZERO_BASE
                    dvec = zvec + SPREAD
                    li[pl.ds(off, 16)] = jnp.where(inb, t, dvec)
                    gi[pl.ds(off, 16)] = jnp.where(inb, t, zvec)

            for stat in range(4):
                dA = pltpu.async_copy(
                    vals_hbm[stat].at[pl.ds(pstart, HALF)], wvA, dsemA)
                dB = pltpu.async_copy(
                    vals_hbm[stat].at[pl.ds(pstart + HALF, HALF)], wvB, dsemB)
                for zc in zcopies:
                    zc.wait()
                plsc.subcore_barrier()
                dA.wait()
                pltpu.sync_copy(wvA, table.at[lidxA], add=True)
                dB.wait()
                pltpu.sync_copy(wvB, table.at[lidxB], add=True)
                plsc.subcore_barrier()

                pltpu.sync_copy(table.at[gidxA], wvA)
                oA = pltpu.async_copy(
                    wvA, outs_hbm[stat].at[pl.ds(ostart, HALF)], dsemA)
                pltpu.sync_copy(table.at[gidxB], wvB)
                plsc.subcore_barrier()
                zcopies = fire_zeros()
                oB = pltpu.async_copy(
                    wvB, outs_hbm[stat].at[pl.ds(ostart + HALF, HALF)], dsemB)
                oA.wait()
                oB.wait()

    for zc in zcopies:
        zc.wait()


def _sc_aggregate(key_enc, m, mz, mr, mg, zer):
    mesh = plsc.VectorSubcoreMesh(core_axis_name="c", subcore_axis_name="s",
                                  num_cores=2, num_subcores=16)
    pt = jax.ShapeDtypeStruct((2 * P,), jnp.float32)
    return pl.kernel(
        _sc_body,
        out_type=[pt] * 4,
        mesh=mesh,
        scratch_types=[
            pltpu.VMEM((HALF,), jnp.int32),
            pltpu.VMEM((HALF,), jnp.int32),
            pltpu.VMEM((HALF,), jnp.int32),
            pltpu.VMEM((HALF,), jnp.int32),
            pltpu.VMEM((HALF,), jnp.float32),
            pltpu.VMEM((HALF,), jnp.float32),
            pltpu.VMEM((ZROWS,), jnp.float32),
            pltpu.VMEM_SHARED((R_TOTAL,), jnp.float32),
            pltpu.SemaphoreType.DMA,
            pltpu.SemaphoreType.DMA,
            pltpu.SemaphoreType.DMA,
        ],
    )(key_enc, m, mz, mr, mg, zer)


_RF = 256
_NBF = R2 // _RF
_LOG32 = math.log(32.0)


def _score_body(c_ref, z_ref, r_ref, g_ref, m_ref,
                score_ref, pmin_ref, pmax_ref):
    cnt = c_ref[0] + c_ref[1]
    zs = z_ref[0] + z_ref[1]
    rs = r_ref[0] + r_ref[1]
    gs = g_ref[0] + g_ref[1]
    cc = jnp.maximum(cnt, 1.0)
    inv = 1.0 / cc
    z_mean = zs * inv
    rel_mean = rs * inv
    geom_mean = gs * inv
    density = jnp.log1p(cnt) / _LOG32
    range_score = jnp.exp(-0.08 * z_mean)
    vs = jnp.maximum(
        (0.55 * geom_mean + 0.25 * density + 0.2 * range_score) * rel_mean,
        0.0)
    score = vs * m_ref[0]
    score_ref[0] = score
    mn = jnp.min(score, axis=0)
    mx = jnp.max(score, axis=0)
    pmin_ref[0, 0] = jnp.broadcast_to(mn[None, :], (8, 128))
    pmax_ref[0, 0] = jnp.broadcast_to(mx[None, :], (8, 128))


def _score(sums, m):
    hs = pl.BlockSpec((2, _RF, 128), lambda n: (0, n, 0))
    ms = pl.BlockSpec((1, _RF, 128), lambda n: (0, n, 0))
    return pl.pallas_call(
        _score_body,
        grid=(_NBF,),
        in_specs=[hs, hs, hs, hs, ms],
        out_specs=[
            pl.BlockSpec((1, _RF, 128), lambda n: (0, n, 0)),
            pl.BlockSpec((1, 1, 8, 128), lambda n: (0, n, 0, 0)),
            pl.BlockSpec((1, 1, 8, 128), lambda n: (0, n, 0, 0)),
        ],
        out_shape=[
            jax.ShapeDtypeStruct((1, R2, 128), jnp.float32),
            jax.ShapeDtypeStruct((1, _NBF, 8, 128), jnp.float32),
            jax.ShapeDtypeStruct((1, _NBF, 8, 128), jnp.float32),
        ],
    )(*sums, m)


def _norm_body(score_ref, pmin_ref, pmax_ref, out_ref):
    mn = jnp.min(pmin_ref[0])
    mx = jnp.max(pmax_ref[0])
    out_ref[0] = (score_ref[0] - mn) / (mx - mn + 1e-6)


def _norm(score, pmin, pmax):
    return pl.pallas_call(
        _norm_body,
        grid=(_NBF,),
        in_specs=[
            pl.BlockSpec((1, _RF, 128), lambda n: (0, n, 0)),
            pl.BlockSpec((1, _NBF, 8, 128), lambda n: (0, 0, 0, 0)),
            pl.BlockSpec((1, _NBF, 8, 128), lambda n: (0, 0, 0, 0)),
        ],
        out_specs=pl.BlockSpec((1, _RF, 128), lambda n: (0, n, 0)),
        out_shape=jax.ShapeDtypeStruct((1, R2, 128), jnp.float32),
    )(score, pmin, pmax)


def kernel(xyz, reliability, geom_break):
    xyz_f = xyz.reshape(B, 3, R2, 128)
    rel_f = reliability.reshape(B, R2, 128)
    geom_f = geom_break.reshape(B, R2, 128)

    zer = jnp.zeros((ZROWS,), jnp.float32)

    outs = []
    for off in range(B):
        key_enc, m, mz, mr, mg = _prep(xyz_f, rel_f, geom_f, off)
        sums = _sc_aggregate(
            key_enc.reshape(P), m.reshape(P), mz.reshape(P),
            mr.reshape(P), mg.reshape(P), zer)
        score, pmin, pmax = _score(
            [s.reshape(2, R2, 128) for s in sums], m)
        outs.append(_norm(score, pmin, pmax))

    return jnp.concatenate(outs, axis=0).reshape(B, 1, 512, 512)
